# baseline (speedup 1.0000x reference)
import functools

import jax
import jax.numpy as jnp
from jax import lax
from jax.experimental import pallas as pl
from jax.experimental.pallas import tpu as pltpu
from jax.experimental.pallas import tpu_sc as plsc

B = 1024
F = 128
N = 100000
BN = 2048
NBM = 98304 // BN
NT = N - NBM * BN

IBIG = 2**31 - 1


CH = 32


def _lex_lt(av, ai, bv, bi):
    return (av < bv) | ((av == bv) & (ai < bi))


def _merge_top2(r1v, r1i, r2v, r2i, c1v, c1i, c2v, c2i):
    take_r = _lex_lt(r1v, r1i, c1v, c1i)
    n1v = jnp.where(take_r, r1v, c1v)
    n1i = jnp.where(take_r, r1i, c1i)
    lv = jnp.where(take_r, c1v, r1v)
    li = jnp.where(take_r, c1i, r1i)
    wv = jnp.where(take_r, r2v, c2v)
    wi = jnp.where(take_r, r2i, c2i)
    t2 = _lex_lt(lv, li, wv, wi)
    return n1v, n1i, jnp.where(t2, lv, wv), jnp.where(t2, li, wi)


def _sims_block(supp_ref, key_ref):
    return lax.dot_general(
        supp_ref[...], key_ref[...],
        dimension_numbers=(((0,), (1,)), ((), ())),
        preferred_element_type=jnp.float32)


def _scan_rows(sims_scr, buf, nchunks, rbase, riota, carry):
    def chunk(c, carry):
        v1, i1, v2, i2 = carry
        xs = sims_scr[buf, pl.ds(c * CH, CH), :]
        for u in range(CH // 8):
            x = lax.slice_in_dim(xs, u * 8, (u + 1) * 8)
            r = riota + (rbase + c * CH + u * 8)
            lt1 = x < v1
            c2 = x < v2
            v2n = jnp.where(c2, x, v2)
            i2n = jnp.where(c2, r, i2)
            v2 = jnp.where(lt1, v1, v2n)
            i2 = jnp.where(lt1, i1, i2n)
            v1 = jnp.where(lt1, x, v1)
            i1 = jnp.where(lt1, r, i1)
        return (v1, i1, v2, i2)

    return lax.fori_loop(0, nchunks, chunk, carry)


def _main_body(supp_ref, key_ref, out_ref, v1s, i1s, v2s, i2s, sims_scr):
    j = pl.program_id(0)

    @pl.when(j == 0)
    def _init():
        sims_scr[1] = jnp.full((BN, B), jnp.inf, jnp.float32)
        v1s[...] = jnp.full((8, B), jnp.inf, jnp.float32)
        i1s[...] = jnp.full((8, B), IBIG, jnp.int32)
        v2s[...] = jnp.full((8, B), jnp.inf, jnp.float32)
        i2s[...] = jnp.full((8, B), IBIG, jnp.int32)

    riota = lax.broadcasted_iota(jnp.int32, (8, B), 0)
    nchunks = jnp.where(j == NBM + 1, NT // CH, BN // CH)
    carry = (v1s[...], i1s[...], v2s[...], i2s[...])
    v1, i1, v2, i2 = _scan_rows(
        sims_scr, (j + 1) % 2, nchunks, (j - 1) * BN, riota, carry)
    v1s[...] = v1
    i1s[...] = i1
    v2s[...] = v2
    i2s[...] = i2

    sims_scr[j % 2] = _sims_block(supp_ref, key_ref)

    @pl.when(j == NBM + 1)
    def _finalize():
        a, b, c, d = v1, i1, v2, i2
        for h in (4, 2, 1):
            a, b, c, d = _merge_top2(
                a[:h], b[:h], c[:h], d[:h],
                a[h:2 * h], b[h:2 * h], c[h:2 * h], d[h:2 * h])
        out_ref[...] = d


def _neighbor_idx(key, support):
    return pl.pallas_call(
        _main_body,
        grid=(NBM + 2,),
        in_specs=[
            pl.BlockSpec((F, BN), lambda j: (0, jnp.minimum(j, NBM))),
            pl.BlockSpec((B, F), lambda j: (0, 0)),
        ],
        out_specs=pl.BlockSpec((1, B), lambda j: (0, 0)),
        out_shape=jax.ShapeDtypeStruct((1, B), jnp.int32),
        scratch_shapes=[
            pltpu.VMEM((8, B), jnp.float32),
            pltpu.VMEM((8, B), jnp.int32),
            pltpu.VMEM((8, B), jnp.float32),
            pltpu.VMEM((8, B), jnp.int32),
            pltpu.VMEM((2, BN, B), jnp.float32),
        ],
    )(support, key)


_NC, _NS, _L = 2, 16, 16
_NW = _NC * _NS
_BPW = B // _NW


def _sc_gather_dot(support_flat, idx, anchor_t):
    mesh = plsc.VectorSubcoreMesh(core_axis_name="c", subcore_axis_name="s")

    nchunk = F * _BPW // 128
    fpc = 128 // _BPW

    @functools.partial(
        pl.kernel,
        mesh=mesh,
        out_type=jax.ShapeDtypeStruct((B,), jnp.float32),
        scratch_types=[
            pltpu.VMEM((_BPW,), jnp.int32),
            pltpu.VMEM((nchunk, 128), jnp.int32),
            pltpu.VMEM((nchunk, 128), jnp.float32),
            pltpu.VMEM((F, _BPW), jnp.float32),
            pltpu.VMEM((_BPW,), jnp.float32),
            pltpu.SemaphoreType.DMA,
        ],
    )
    def k(supp_hbm, idx_hbm, anc_hbm, out_hbm, idx_v, fidx_v, gath_v, anc_v,
          out_v, sem):
        wid = lax.axis_index("s") * _NC + lax.axis_index("c")
        base = wid * _BPW
        pltpu.sync_copy(idx_hbm.at[pl.ds(base, _BPW)], idx_v)
        pltpu.sync_copy(anc_hbm.at[wid], anc_v)
        ia = idx_v[pl.ds(0, _L)]
        ib = idx_v[pl.ds(_L, _L)]

        def fill(c, _):
            for q in range(fpc):
                off = (fpc * c + q) * N
                fidx_v[c, pl.ds(q * _BPW, _L)] = ia + off
                fidx_v[c, pl.ds(q * _BPW + _L, _L)] = ib + off
            return 0

        lax.fori_loop(0, nchunk, fill, 0)

        def fire(c, _):
            pltpu.async_copy(supp_hbm.at[fidx_v.at[c]], gath_v.at[c], sem)
            return 0

        lax.fori_loop(0, nchunk, fire, 0)

        def drain(c, _):
            pltpu.make_async_copy(
                supp_hbm.at[fidx_v.at[c]], gath_v.at[c], sem).wait()
            return 0

        lax.fori_loop(0, nchunk, drain, 0)

        def acc(c, carry):
            a0, a1 = carry
            for q in range(fpc):
                f = fpc * c + q
                a0 = a0 + anc_v[f, pl.ds(0, _L)] * gath_v[c, pl.ds(q * _BPW, _L)]
                a1 = a1 + anc_v[f, pl.ds(_L, _L)] * gath_v[c, pl.ds(q * _BPW + _L, _L)]
            return (a0, a1)

        zero = jnp.zeros((_L,), jnp.float32)
        a0, a1 = lax.fori_loop(0, nchunk, acc, (zero, zero))
        out_v[pl.ds(0, _L)] = a0
        out_v[pl.ds(_L, _L)] = a1
        pltpu.sync_copy(out_v, out_hbm.at[pl.ds(base, _BPW)])

    return k(support_flat, idx, anchor_t)


def kernel(anchor, key, support):
    idx = _neighbor_idx(key, support).reshape(B)
    support_flat = support.reshape(F * N)
    anchor_t = anchor.T.reshape(F, _NW, _BPW).transpose(1, 0, 2)
    out = _sc_gather_dot(support_flat, idx, anchor_t)
    return out.reshape(B, 1)
